# baseline (speedup 1.0000x reference)
import math

import jax
import jax.numpy as jnp
from jax.experimental import pallas as pl
from jax.experimental.pallas import tpu as pltpu

_S = 10
_L2E = math.log2(math.e)


def kernel(gamma, z0, t_init, t_last, event_times, pair_i, pair_j):
    O, N, D = z0.shape
    B = t_init.shape[0]
    E = event_times.shape[1]
    T = B * _S + B * E
    f32 = jnp.float32
    ev_tot = B * E
    inv_fact = [1.0 / math.factorial(o) for o in range(O)]
    RB = 64
    n_chunks = N // RB
    scl = _L2E * _L2E
    TPG = 40
    n_steps = T // TPG

    step = (t_last - t_init) / _S
    tq = t_init[:, None] + step[:, None] * jnp.arange(_S, dtype=f32)[None, :]
    t_all = jnp.concatenate([tq.reshape(-1), event_times.reshape(-1)])
    w_all = jnp.concatenate([jnp.repeat(step, _S), jnp.ones((ev_tot,), f32)])
    flag_all = jnp.concatenate(
        [jnp.ones((B * _S,), jnp.int32), jnp.zeros((ev_tot,), jnp.int32)])

    z0_dn = jnp.transpose(z0, (0, 2, 1))
    g_row = gamma.reshape(1, N)
    g_col = gamma.reshape(N, 1)

    def body(t_ref, w_ref, f_ref, z0dn_ref, grow_ref, gcol_ref,
             out_ref, accq_ref, acce_ref, gsum_ref, trif_ref):
        s = pl.program_id(0)
        flag = f_ref[TPG * s]

        @pl.when(s == 0)
        def _init():
            accq_ref[...] = jnp.zeros_like(accq_ref)
            acce_ref[...] = jnp.zeros_like(acce_ref)
            tri = (jax.lax.broadcasted_iota(jnp.int32, (RB, RB), 1)
                   > jax.lax.broadcasted_iota(jnp.int32, (RB, RB), 0))
            trif_ref[...] = jnp.where(tri, 1.0, 0.0)
            gsum_ref[...] = _L2E * (gcol_ref[...] + grow_ref[...])
            for r in range(n_chunks):
                lo = r * RB
                blk = gsum_ref[lo:lo + RB, lo:lo + RB]
                gsum_ref[lo:lo + RB, lo:lo + RB] = jnp.where(tri, blk, -1e30)

        def make_ops(k):
            t = t_ref[TPG * s + k]
            coefs = []
            p = jnp.float32(1.0)
            for o in range(O):
                coefs.append(p * inv_fact[o])
                p = p * t
            zdn = coefs[0] * z0dn_ref[0]
            for o in range(1, O):
                zdn = zdn + coefs[o] * z0dn_ref[o]
            nrow = jnp.sum(zdn * zdn, axis=0, keepdims=True)
            ones_row = jnp.ones((1, N), f32)
            nrow_s = scl * nrow
            lhsT = jnp.concatenate([zdn, nrow_s, ones_row], axis=0)
            rhs = jnp.concatenate([(-2.0 * scl) * zdn, ones_row, nrow_s],
                                  axis=0)
            return lhsT, rhs

        def dist_chunk(ops, r):
            lhsT, rhs = ops
            lo = r * RB
            d2 = jax.lax.dot_general(
                lhsT[:, lo:lo + RB], rhs[:, lo:],
                (((0,), (0,)), ((), ())),
                preferred_element_type=f32)
            d2c = jnp.maximum(d2, 1e-12)
            return d2c * jax.lax.rsqrt(d2c)

        def fold8(vals):
            return jnp.sum(vals.reshape(RB // 8, 8, vals.shape[1]), axis=0)

        @pl.when(flag != 0)
        def _quad():
            ops = [make_ops(k) for k in range(TPG)]
            ws = [w_ref[TPG * s + k] for k in range(TPG)]
            for r in range(n_chunks):
                lo = r * RB
                gs = gsum_ref[lo:lo + RB, lo:]
                upd = accq_ref[:, lo:]
                for k in range(TPG):
                    m = jnp.exp2(gs - dist_chunk(ops[k], r))
                    upd = upd + ws[k] * fold8(m)
                accq_ref[:, lo:] = upd

        @pl.when(flag == 0)
        def _event():
            ops = [make_ops(k) for k in range(TPG)]
            for r in range(n_chunks):
                lo = r * RB
                updh = acce_ref[:, lo:lo + RB]
                updt = acce_ref[:, lo + RB:] if lo + RB < N else None
                for k in range(TPG):
                    dist = dist_chunk(ops[k], r)
                    updh = updh + fold8(dist[:, :RB] * trif_ref[...])
                    if updt is not None:
                        updt = updt + fold8(dist[:, RB:])
                acce_ref[:, lo:lo + RB] = updh
                if updt is not None:
                    acce_ref[:, lo + RB:] = updt

        @pl.when(s == n_steps - 1)
        def _fin():
            const = -(ev_tot * (N - 1)) * jnp.sum(grow_ref[...])
            total = (const + jnp.sum(accq_ref[...])
                     + jnp.sum(acce_ref[...]) * (1.0 / _L2E))
            out_ref[...] = jnp.zeros_like(out_ref) + total

    grid_spec = pltpu.PrefetchScalarGridSpec(
        num_scalar_prefetch=3,
        grid=(n_steps,),
        in_specs=[
            pl.BlockSpec((O, D, N), lambda s, *_: (0, 0, 0)),
            pl.BlockSpec((1, N), lambda s, *_: (0, 0)),
            pl.BlockSpec((N, 1), lambda s, *_: (0, 0)),
        ],
        out_specs=pl.BlockSpec((1, 128), lambda s, *_: (0, 0)),
        scratch_shapes=[
            pltpu.VMEM((8, N), jnp.float32),
            pltpu.VMEM((8, N), jnp.float32),
            pltpu.VMEM((N, N), jnp.float32),
            pltpu.VMEM((RB, RB), jnp.float32),
        ],
    )
    out = pl.pallas_call(
        body,
        grid_spec=grid_spec,
        out_shape=jax.ShapeDtypeStruct((1, 128), f32),
        compiler_params=pltpu.CompilerParams(
            dimension_semantics=("arbitrary",),
        ),
        name="nhpp_nll",
    )(t_all, w_all, flag_all, z0_dn, g_row, g_col)
    return out[0, 0]


# speedup vs baseline: 4.1657x; 4.1657x over previous
import math

import jax
import jax.numpy as jnp
from jax.experimental import pallas as pl
from jax.experimental.pallas import tpu as pltpu

_S = 10
_L2E = math.log2(math.e)


def kernel(gamma, z0, t_init, t_last, event_times, pair_i, pair_j):
    O, N, D = z0.shape
    B = t_init.shape[0]
    E = event_times.shape[1]
    T = B * _S + B * E
    f32 = jnp.float32
    ev_tot = B * E
    inv_fact = [1.0 / math.factorial(o) for o in range(O)]
    RB = 128
    n_chunks = N // RB
    scl = _L2E * _L2E
    TPG = 40
    n_steps = T // TPG

    step = (t_last - t_init) / _S
    tq = t_init[:, None] + step[:, None] * jnp.arange(_S, dtype=f32)[None, :]
    t_all = jnp.concatenate([tq.reshape(-1), event_times.reshape(-1)])
    w_all = jnp.concatenate([jnp.repeat(step, _S), jnp.ones((ev_tot,), f32)])
    flag_all = jnp.concatenate(
        [jnp.ones((B * _S,), jnp.int32), jnp.zeros((ev_tot,), jnp.int32)])

    z0_dn = jnp.transpose(z0, (0, 2, 1))
    g_row = gamma.reshape(1, N)
    g_col = gamma.reshape(N, 1)

    def body(t_ref, w_ref, f_ref, z0dn_ref, grow_ref, gcol_ref,
             out_ref, accq_ref, acce_ref, gsum_ref, trif_ref):
        s = pl.program_id(0)
        flag = f_ref[TPG * s]

        @pl.when(s == 0)
        def _init():
            accq_ref[...] = jnp.zeros_like(accq_ref)
            acce_ref[...] = jnp.zeros_like(acce_ref)
            tri = (jax.lax.broadcasted_iota(jnp.int32, (RB, RB), 1)
                   > jax.lax.broadcasted_iota(jnp.int32, (RB, RB), 0))
            trif_ref[...] = jnp.where(tri, 1.0, 0.0).astype(jnp.bfloat16)
            gs_full = _L2E * (gcol_ref[...] + grow_ref[...])
            gsum_ref[...] = gs_full.astype(jnp.bfloat16)
            for r in range(n_chunks):
                lo = r * RB
                blk = gsum_ref[lo:lo + RB, lo:lo + RB]
                gsum_ref[lo:lo + RB, lo:lo + RB] = jnp.where(
                    tri, blk, jnp.bfloat16(-1e30))

        def make_ops(k):
            t = t_ref[TPG * s + k]
            coefs = []
            p = jnp.float32(1.0)
            for o in range(O):
                coefs.append(p * inv_fact[o])
                p = p * t
            zdn = coefs[0] * z0dn_ref[0]
            for o in range(1, O):
                zdn = zdn + coefs[o] * z0dn_ref[o]
            nrow = jnp.sum(zdn * zdn, axis=0, keepdims=True)
            ones_row = jnp.ones((1, N), f32)
            nrow_s = scl * nrow
            lhsT = jnp.concatenate([zdn, nrow_s, ones_row], axis=0)
            rhs = jnp.concatenate([(-2.0 * scl) * zdn, ones_row, nrow_s],
                                  axis=0)
            return lhsT, rhs

        def dist_chunk(ops, r):
            lhsT, rhs = ops
            lo = r * RB
            d2 = jax.lax.dot_general(
                lhsT[:, lo:lo + RB], rhs[:, lo:],
                (((0,), (0,)), ((), ())),
                preferred_element_type=f32)
            d2c = jnp.maximum(d2, 1e-12).astype(jnp.bfloat16)
            return d2c * jax.lax.rsqrt(d2c)

        def fold16(vals):
            r16 = jnp.sum(vals.reshape(RB // 16, 16, vals.shape[1]), axis=0)
            return r16.astype(f32)

        @pl.when(flag != 0)
        def _quad():
            ops = [make_ops(k) for k in range(TPG)]
            ws = [w_ref[TPG * s + k] for k in range(TPG)]
            for r in range(n_chunks):
                lo = r * RB
                gs = gsum_ref[lo:lo + RB, lo:]
                upd = accq_ref[:, lo:]
                for k in range(TPG):
                    m = jnp.exp2(gs - dist_chunk(ops[k], r))
                    upd = upd + ws[k] * fold16(m)
                accq_ref[:, lo:] = upd

        @pl.when(flag == 0)
        def _event():
            ops = [make_ops(k) for k in range(TPG)]
            for r in range(n_chunks):
                lo = r * RB
                updh = acce_ref[:, lo:lo + RB]
                updt = acce_ref[:, lo + RB:] if lo + RB < N else None
                for k in range(TPG):
                    dist = dist_chunk(ops[k], r)
                    updh = updh + fold16(dist[:, :RB] * trif_ref[...])
                    if updt is not None:
                        updt = updt + fold16(dist[:, RB:])
                acce_ref[:, lo:lo + RB] = updh
                if updt is not None:
                    acce_ref[:, lo + RB:] = updt

        @pl.when(s == n_steps - 1)
        def _fin():
            const = -(ev_tot * (N - 1)) * jnp.sum(grow_ref[...])
            total = (const + jnp.sum(accq_ref[...])
                     + jnp.sum(acce_ref[...]) * (1.0 / _L2E))
            out_ref[...] = jnp.zeros_like(out_ref) + total

    grid_spec = pltpu.PrefetchScalarGridSpec(
        num_scalar_prefetch=3,
        grid=(n_steps,),
        in_specs=[
            pl.BlockSpec((O, D, N), lambda s, *_: (0, 0, 0)),
            pl.BlockSpec((1, N), lambda s, *_: (0, 0)),
            pl.BlockSpec((N, 1), lambda s, *_: (0, 0)),
        ],
        out_specs=pl.BlockSpec((1, 128), lambda s, *_: (0, 0)),
        scratch_shapes=[
            pltpu.VMEM((16, N), jnp.float32),
            pltpu.VMEM((16, N), jnp.float32),
            pltpu.VMEM((N, N), jnp.bfloat16),
            pltpu.VMEM((RB, RB), jnp.bfloat16),
        ],
    )
    out = pl.pallas_call(
        body,
        grid_spec=grid_spec,
        out_shape=jax.ShapeDtypeStruct((1, 128), f32),
        compiler_params=pltpu.CompilerParams(
            dimension_semantics=("arbitrary",),
        ),
        name="nhpp_nll",
    )(t_all, w_all, flag_all, z0_dn, g_row, g_col)
    return out[0, 0]


# speedup vs baseline: 4.4555x; 1.0695x over previous
import math

import jax
import jax.numpy as jnp
from jax.experimental import pallas as pl
from jax.experimental.pallas import tpu as pltpu

_S = 10
_L2E = math.log2(math.e)


def kernel(gamma, z0, t_init, t_last, event_times, pair_i, pair_j):
    O, N, D = z0.shape
    B = t_init.shape[0]
    E = event_times.shape[1]
    T = B * _S + B * E
    f32 = jnp.float32
    ev_tot = B * E
    inv_fact = [1.0 / math.factorial(o) for o in range(O)]
    RB = 128
    n_chunks = N // RB
    scl = _L2E * _L2E
    TPG = 40
    n_steps = T // TPG

    step = (t_last - t_init) / _S
    tq = t_init[:, None] + step[:, None] * jnp.arange(_S, dtype=f32)[None, :]
    t_all = jnp.concatenate([tq.reshape(-1), event_times.reshape(-1)])
    w_all = jnp.concatenate([jnp.repeat(step, _S), jnp.ones((ev_tot,), f32)])
    flag_all = jnp.concatenate(
        [jnp.ones((B * _S,), jnp.int32), jnp.zeros((ev_tot,), jnp.int32)])

    z0_dn = jnp.transpose(z0, (0, 2, 1))
    g_row = gamma.reshape(1, N)
    g_col = gamma.reshape(N, 1)

    def body(t_ref, w_ref, f_ref, z0dn_ref, grow_ref, gcol_ref,
             out_ref, accq_ref, acce_ref, gsum_ref, trif_ref):
        s = pl.program_id(0)
        flag = f_ref[TPG * s]

        @pl.when(s == 0)
        def _init():
            accq_ref[...] = jnp.zeros_like(accq_ref)
            acce_ref[...] = jnp.zeros_like(acce_ref)
            tri = (jax.lax.broadcasted_iota(jnp.int32, (RB, RB), 1)
                   > jax.lax.broadcasted_iota(jnp.int32, (RB, RB), 0))
            trif_ref[...] = jnp.where(tri, 1.0, 0.0)
            gsum_ref[...] = _L2E * (gcol_ref[...] + grow_ref[...])
            for r in range(n_chunks):
                lo = r * RB
                blk = gsum_ref[lo:lo + RB, lo:lo + RB]
                gsum_ref[lo:lo + RB, lo:lo + RB] = jnp.where(tri, blk, -1e30)

        def make_ops(k):
            t = t_ref[TPG * s + k]
            coefs = []
            p = jnp.float32(1.0)
            for o in range(O):
                coefs.append(p * inv_fact[o])
                p = p * t
            zdn = coefs[0] * z0dn_ref[0]
            for o in range(1, O):
                zdn = zdn + coefs[o] * z0dn_ref[o]
            nrow = jnp.sum(zdn * zdn, axis=0, keepdims=True)
            ones_row = jnp.ones((1, N), f32)
            nrow_s = scl * nrow
            lhsT = jnp.concatenate([zdn, nrow_s, ones_row], axis=0)
            rhs = jnp.concatenate([(-2.0 * scl) * zdn, ones_row, nrow_s],
                                  axis=0)
            return lhsT, rhs

        def dist_chunk(ops, r):
            lhsT, rhs = ops
            lo = r * RB
            d2 = jax.lax.dot_general(
                lhsT[:, lo:lo + RB], rhs[:, lo:],
                (((0,), (0,)), ((), ())),
                preferred_element_type=f32)
            d2c = jnp.maximum(d2, 1e-12)
            return d2c * jax.lax.rsqrt(d2c)

        def fold8(vals):
            return jnp.sum(vals.reshape(RB // 8, 8, vals.shape[1]), axis=0)

        @pl.when(flag != 0)
        def _quad():
            ops = [make_ops(k) for k in range(TPG)]
            ws = [w_ref[TPG * s + k] for k in range(TPG)]
            for r in range(n_chunks):
                lo = r * RB
                gs = gsum_ref[lo:lo + RB, lo:]
                upd = accq_ref[:, lo:]
                for k in range(TPG):
                    m = jnp.exp2(gs - dist_chunk(ops[k], r))
                    upd = upd + ws[k] * fold8(m)
                accq_ref[:, lo:] = upd

        @pl.when(flag == 0)
        def _event():
            ops = [make_ops(k) for k in range(TPG)]
            for r in range(n_chunks):
                lo = r * RB
                updh = acce_ref[:, lo:lo + RB]
                updt = acce_ref[:, lo + RB:] if lo + RB < N else None
                for k in range(TPG):
                    dist = dist_chunk(ops[k], r)
                    updh = updh + fold8(dist[:, :RB] * trif_ref[...])
                    if updt is not None:
                        updt = updt + fold8(dist[:, RB:])
                acce_ref[:, lo:lo + RB] = updh
                if updt is not None:
                    acce_ref[:, lo + RB:] = updt

        @pl.when(s == n_steps - 1)
        def _fin():
            const = -(ev_tot * (N - 1)) * jnp.sum(grow_ref[...])
            total = (const + jnp.sum(accq_ref[...])
                     + jnp.sum(acce_ref[...]) * (1.0 / _L2E))
            out_ref[...] = jnp.zeros_like(out_ref) + total

    grid_spec = pltpu.PrefetchScalarGridSpec(
        num_scalar_prefetch=3,
        grid=(n_steps,),
        in_specs=[
            pl.BlockSpec((O, D, N), lambda s, *_: (0, 0, 0)),
            pl.BlockSpec((1, N), lambda s, *_: (0, 0)),
            pl.BlockSpec((N, 1), lambda s, *_: (0, 0)),
        ],
        out_specs=pl.BlockSpec((1, 128), lambda s, *_: (0, 0)),
        scratch_shapes=[
            pltpu.VMEM((8, N), jnp.float32),
            pltpu.VMEM((8, N), jnp.float32),
            pltpu.VMEM((N, N), jnp.float32),
            pltpu.VMEM((RB, RB), jnp.float32),
        ],
    )
    out = pl.pallas_call(
        body,
        grid_spec=grid_spec,
        out_shape=jax.ShapeDtypeStruct((1, 128), f32),
        compiler_params=pltpu.CompilerParams(
            dimension_semantics=("arbitrary",),
        ),
        name="nhpp_nll",
    )(t_all, w_all, flag_all, z0_dn, g_row, g_col)
    return out[0, 0]
